# baseline (speedup 1.0000x reference)
import functools

import jax
import jax.numpy as jnp
from jax import lax
from jax.experimental import pallas as pl
from jax.experimental.pallas import tpu as pltpu
from jax.experimental.pallas import tpu_sc as plsc

BATCH = 16384
EMBED_DIM = 64
CLS_DIM = 32
NC = 2
NS = 16
NW = NC * NS
B_PER_W = BATCH // NW
CHUNK = 128
N_CHUNK = B_PER_W // CHUNK


@functools.partial(
    pl.kernel,
    out_type=(
        jax.ShapeDtypeStruct((BATCH, EMBED_DIM), jnp.float32),
        jax.ShapeDtypeStruct((BATCH, CLS_DIM), jnp.float32),
    ),
    mesh=plsc.VectorSubcoreMesh(core_axis_name="c", subcore_axis_name="s"),
    compiler_params=pltpu.CompilerParams(use_tc_tiling_on_sc=False),
    scratch_types=[
        pltpu.VMEM((N_CHUNK, CHUNK), jnp.int32),
        pltpu.VMEM((N_CHUNK, CHUNK), jnp.int32),
        pltpu.VMEM((B_PER_W, EMBED_DIM), jnp.float32),
        pltpu.VMEM((B_PER_W, CLS_DIM), jnp.float32),
        pltpu.SemaphoreType.DMA,
    ],
)
def _sc_gather(ids_hbm, cls_hbm, emb_hbm, clsemb_hbm, act_out, cls_out,
               ids_v, clsids_v, act_rows, cls_rows, sem):
    wid = lax.axis_index("s") * NC + lax.axis_index("c")
    base = wid * B_PER_W
    pltpu.sync_copy(ids_hbm.at[wid], ids_v)
    pltpu.sync_copy(cls_hbm.at[wid], clsids_v)
    copies = []
    for j in range(N_CHUNK):
        copies.append(pltpu.async_copy(
            emb_hbm.at[ids_v.at[j]], act_rows.at[pl.ds(j * CHUNK, CHUNK)], sem))
        copies.append(pltpu.async_copy(
            clsemb_hbm.at[clsids_v.at[j]], cls_rows.at[pl.ds(j * CHUNK, CHUNK)], sem))
    for c in copies:
        c.wait()
    pltpu.sync_copy(act_rows, act_out.at[pl.ds(base, B_PER_W)])
    pltpu.sync_copy(cls_rows, cls_out.at[pl.ds(base, B_PER_W)])


def _mm_body(act_ref, cls_ref, w1_ref, w2_ref, b_ref, o_ref):
    acc = jnp.dot(act_ref[...], w1_ref[...],
                  preferred_element_type=jnp.float32,
                  precision=lax.Precision.HIGHEST)
    acc += jnp.dot(cls_ref[...], w2_ref[...],
                   preferred_element_type=jnp.float32,
                   precision=lax.Precision.HIGHEST)
    o_ref[...] = acc + b_ref[...]


def _tc_project(act_emb, cls_emb, w1, w2, b2d):
    blk = 2048
    grid = (BATCH // blk,)
    return pl.pallas_call(
        _mm_body,
        grid=grid,
        in_specs=[
            pl.BlockSpec((blk, EMBED_DIM), lambda i: (i, 0)),
            pl.BlockSpec((blk, CLS_DIM), lambda i: (i, 0)),
            pl.BlockSpec((EMBED_DIM, EMBED_DIM), lambda i: (0, 0)),
            pl.BlockSpec((CLS_DIM, EMBED_DIM), lambda i: (0, 0)),
            pl.BlockSpec((1, EMBED_DIM), lambda i: (0, 0)),
        ],
        out_specs=pl.BlockSpec((blk, EMBED_DIM), lambda i: (i, 0)),
        out_shape=jax.ShapeDtypeStruct((BATCH, EMBED_DIM), jnp.float32),
    )(act_emb, cls_emb, w1, w2, b2d)


def kernel(activity_ids, activity_classes, embedding, class_embedding, W, b):
    ids = activity_ids.astype(jnp.int32).reshape(NW, N_CHUNK, CHUNK)
    cls = activity_classes.astype(jnp.int32).reshape(NW, N_CHUNK, CHUNK)
    act_emb, cls_emb = _sc_gather(ids, cls, embedding, class_embedding)
    return _tc_project(act_emb, cls_emb,
                       W[:EMBED_DIM], W[EMBED_DIM:], b.reshape(1, EMBED_DIM))
